# baseline (speedup 1.0000x reference)
import functools

import jax
import jax.numpy as jnp
from jax import lax
from jax.experimental import pallas as pl
from jax.experimental.pallas import tpu as pltpu
from jax.experimental.pallas import tpu_sc as plsc

NN = 10000
EE = 320000
DD = 128
HH = 64
NC, NS, LL = 2, 16, 16
NWORK = NC * NS
CHUNK = 128
NCH = 80
NBUF = 2
NGRP = NCH // NBUF
EPAD = NWORK * NCH * CHUNK
ROWS_PER_TILE = 632
NP = NS * ROWS_PER_TILE
EW = EE // NWORK

_MESH = plsc.VectorSubcoreMesh(
    core_axis_name="c", subcore_axis_name="s", num_cores=NC, num_subcores=NS)


def _wid():
  return lax.axis_index("c") * NS + lax.axis_index("s")


@functools.partial(
    pl.kernel,
    out_type=jax.ShapeDtypeStruct((NP,), jnp.float32),
    mesh=_MESH,
    compiler_params=pltpu.CompilerParams(
        use_tc_tiling_on_sc=False, needs_layout_passes=False),
    scratch_types=[
        pltpu.VMEM((2 * NCH, CHUNK), jnp.int32),
        pltpu.VMEM((NP,), jnp.float32),
        pltpu.VMEM((640,), jnp.float32),
        pltpu.VMEM((640,), jnp.float32),
        pltpu.VMEM((640,), jnp.float32),
        pltpu.VMEM_SHARED((NS, NP), jnp.float32),
    ],
)
def _sc_degree(dst3, dinv_out, dst_l, cnt_l, dinv_l, accb, tbuf, stage):
  c = lax.axis_index("c")
  s = lax.axis_index("s")
  rbase = s * ROWS_PER_TILE
  rpt = pl.ds(rbase, ROWS_PER_TILE)
  pltpu.sync_copy(dst3.at[s], dst_l.at[pl.ds(0, NCH)])
  pltpu.sync_copy(dst3.at[s + NS], dst_l.at[pl.ds(NCH, NCH)])

  @pl.loop(0, NP // LL)
  def _(i):
    cnt_l[pl.ds(i * LL, LL)] = jnp.zeros((LL,), jnp.float32)

  ones = jnp.ones((LL,), jnp.float32)

  @pl.loop(0, 2 * NCH)
  def _(j):
    for k in range(CHUNK // LL):
      dv = dst_l[j, pl.ds(k * LL, LL)]
      plsc.addupdate_scatter(cnt_l, [dv], ones)

  pltpu.sync_copy(cnt_l, stage.at[s])
  plsc.subcore_barrier()

  @pl.loop(0, 640 // LL)
  def _(i):
    accb[pl.ds(i * LL, LL)] = jnp.zeros((LL,), jnp.float32)

  for t in range(NS):
    pltpu.sync_copy(stage.at[t, rpt], tbuf.at[pl.ds(0, ROWS_PER_TILE)])

    @pl.loop(0, 640 // LL)
    def _(i):
      sl = pl.ds(i * LL, LL)
      accb[sl] = accb[sl] + tbuf[sl]

  @pl.loop(0, 640 // LL)
  def _(i):
    sl = pl.ds(i * LL, LL)
    d = accb[sl] + 1.0
    bits = plsc.bitcast(d, jnp.int32)
    y = plsc.bitcast(0x5F3759DF - lax.shift_right_logical(bits, 1),
                     jnp.float32)
    y = y * (1.5 - 0.5 * d * y * y)
    y = y * (1.5 - 0.5 * d * y * y)
    y = y * (1.5 - 0.5 * d * y * y)
    dinv_l[sl] = y

  @pl.when(c == 0)
  def _():
    pltpu.sync_copy(dinv_l.at[pl.ds(0, ROWS_PER_TILE)], dinv_out.at[rpt])


@functools.partial(
    pl.kernel,
    out_type=jax.ShapeDtypeStruct((NC, NP, HH), jnp.float32),
    mesh=_MESH,
    compiler_params=pltpu.CompilerParams(
        use_tc_tiling_on_sc=False, needs_layout_passes=False),
    scratch_types=[
        pltpu.VMEM((NCH, CHUNK), jnp.int32),
        pltpu.VMEM((NCH, CHUNK), jnp.int32),
        pltpu.VMEM((CHUNK, HH), jnp.float32),
        pltpu.VMEM((CHUNK, HH), jnp.float32),
        pltpu.SemaphoreType.DMA,
        pltpu.SemaphoreType.DMA,
        pltpu.VMEM_SHARED((NP, HH), jnp.float32),
        pltpu.VMEM_SHARED((NP, HH), jnp.float32),
    ],
)
def _sc_msgpass(g1p, src3, dst3, zeros64, s_out,
                src_l, dst_l, rows0, rows1, sem0, sem1, g_sp, acc):
  c = lax.axis_index("c")
  s = lax.axis_index("s")
  w = _wid()
  rbase = s * ROWS_PER_TILE
  rpt = pl.ds(rbase, ROWS_PER_TILE)
  pltpu.sync_copy(src3.at[w], src_l)
  pltpu.sync_copy(dst3.at[w], dst_l)
  pltpu.sync_copy(g1p.at[rpt], g_sp.at[rpt])
  pltpu.sync_copy(zeros64.at[rpt], acc.at[rpt])
  plsc.subcore_barrier()

  pltpu.async_copy(g_sp.at[src_l.at[0]], rows0, sem0)

  @pl.loop(0, NCH - 1)
  def _(j):
    even = j % 2 == 0

    def do(cur, nxt, sem_cur, sem_nxt):
      pltpu.async_copy(g_sp.at[src_l.at[j + 1]], nxt, sem_nxt)
      pltpu.make_async_copy(g_sp.at[src_l.at[j]], cur, sem_cur).wait()
      pltpu.sync_copy(cur, acc.at[dst_l.at[j]], add=True)

    @pl.when(even)
    def _():
      do(rows0, rows1, sem0, sem1)

    @pl.when(jnp.logical_not(even))
    def _():
      do(rows1, rows0, sem1, sem0)

  last = NCH - 1
  pltpu.make_async_copy(g_sp.at[src_l.at[last]], rows1, sem1).wait()
  pltpu.sync_copy(rows1, acc.at[dst_l.at[last]], add=True)

  plsc.subcore_barrier()
  pltpu.sync_copy(acc.at[rpt], s_out.at[c, rpt])


@functools.partial(
    pl.kernel,
    out_type=jax.ShapeDtypeStruct((NWORK, NCH * CHUNK), jnp.float32),
    mesh=_MESH,
    compiler_params=pltpu.CompilerParams(
        use_tc_tiling_on_sc=False, needs_layout_passes=False),
    scratch_types=[
        pltpu.VMEM((2 * NCH, CHUNK), jnp.int32),
        pltpu.VMEM((2 * NCH, CHUNK), jnp.int32),
        pltpu.VMEM((NP,), jnp.float32),
        pltpu.VMEM((NP,), jnp.float32),
        pltpu.VMEM((NP,), jnp.float32),
        pltpu.VMEM((NP,), jnp.float32),
        pltpu.VMEM((640,), jnp.float32),
        pltpu.VMEM((640,), jnp.float32),
        pltpu.VMEM((640,), jnp.float32),
        pltpu.VMEM((640,), jnp.float32),
        pltpu.VMEM((640,), jnp.float32),
        pltpu.VMEM((640,), jnp.float32),
        pltpu.VMEM((NP,), jnp.float32),
        pltpu.VMEM((NP,), jnp.float32),
        pltpu.VMEM((NCH * CHUNK,), jnp.float32),
        pltpu.VMEM_SHARED((NS, NP), jnp.float32),
        pltpu.VMEM_SHARED((NP,), jnp.float32),
        pltpu.VMEM_SHARED((NP,), jnp.float32),
    ],
)
def _sc_scalar_decode(p_p, q_p, src3, dst3, dinv_p, us_p, ut_p,
                      dec_out,
                      src_l, dst_l, p_l, q_l, pacc, qacc,
                      dinv_l, us_l, ut_l, accb, tbuf, sv_l,
                      s_l, t_l, ob,
                      stage, s_sp, t_sp):
  c = lax.axis_index("c")
  s = lax.axis_index("s")
  rbase = s * ROWS_PER_TILE
  rpt = pl.ds(rbase, ROWS_PER_TILE)
  pltpu.sync_copy(p_p, p_l)
  pltpu.sync_copy(q_p, q_l)
  pltpu.sync_copy(src3.at[s], src_l.at[pl.ds(0, NCH)])
  pltpu.sync_copy(src3.at[s + NS], src_l.at[pl.ds(NCH, NCH)])
  pltpu.sync_copy(dst3.at[s], dst_l.at[pl.ds(0, NCH)])
  pltpu.sync_copy(dst3.at[s + NS], dst_l.at[pl.ds(NCH, NCH)])
  pltpu.sync_copy(dinv_p.at[rpt], dinv_l.at[pl.ds(0, ROWS_PER_TILE)])
  pltpu.sync_copy(us_p.at[rpt], us_l.at[pl.ds(0, ROWS_PER_TILE)])
  pltpu.sync_copy(ut_p.at[rpt], ut_l.at[pl.ds(0, ROWS_PER_TILE)])

  @pl.loop(0, NP // LL)
  def _(i):
    z = jnp.zeros((LL,), jnp.float32)
    pacc[pl.ds(i * LL, LL)] = z
    qacc[pl.ds(i * LL, LL)] = z

  @pl.loop(0, 2 * NCH)
  def _(j):
    for k in range(CHUNK // LL):
      sv = src_l[j, pl.ds(k * LL, LL)]
      dv = dst_l[j, pl.ds(k * LL, LL)]
      pv = plsc.load_gather(p_l, [sv])
      qv = plsc.load_gather(q_l, [sv])
      plsc.addupdate_scatter(pacc, [dv], pv)
      plsc.addupdate_scatter(qacc, [dv], qv)

  def reduce_into(ul, dst_sp):
    @pl.loop(0, 640 // LL)
    def _(i):
      accb[pl.ds(i * LL, LL)] = jnp.zeros((LL,), jnp.float32)

    for t in range(NS):
      pltpu.sync_copy(stage.at[t, rpt], tbuf.at[pl.ds(0, ROWS_PER_TILE)])

      @pl.loop(0, 640 // LL)
      def _(i):
        sl = pl.ds(i * LL, LL)
        accb[sl] = accb[sl] + tbuf[sl]

    @pl.loop(0, 640 // LL)
    def _(i):
      sl = pl.ds(i * LL, LL)
      sv_l[sl] = dinv_l[sl] * accb[sl] + ul[sl]

    pltpu.sync_copy(sv_l.at[pl.ds(0, ROWS_PER_TILE)], dst_sp.at[rpt])

  pltpu.sync_copy(pacc, stage.at[s])
  plsc.subcore_barrier()
  reduce_into(us_l, s_sp)
  plsc.subcore_barrier()
  pltpu.sync_copy(qacc, stage.at[s])
  plsc.subcore_barrier()
  reduce_into(ut_l, t_sp)
  plsc.subcore_barrier()

  pltpu.sync_copy(s_sp, s_l)
  pltpu.sync_copy(t_sp, t_l)
  w = _wid()
  wrow = c * NCH

  @pl.loop(0, NCH)
  def _(j):
    for k in range(CHUNK // LL):
      si = src_l[wrow + j, pl.ds(k * LL, LL)]
      di = dst_l[wrow + j, pl.ds(k * LL, LL)]
      sv = plsc.load_gather(s_l, [si])
      tv = plsc.load_gather(t_l, [di])
      y = sv + tv
      ob[pl.ds(j * CHUNK + k * LL, LL)] = 1.0 / (1.0 + jnp.exp(-y))

  pltpu.sync_copy(ob, dec_out.at[w])


def _tc_h1_body(x_ref, w1_ref, dinv_ref, h1_ref, g1_ref):
  h = jnp.dot(x_ref[...], w1_ref[...], preferred_element_type=jnp.float32)
  zt = jnp.zeros((NP - NN, HH), jnp.float32)
  h1_ref[...] = jnp.concatenate([h, zt], axis=0)
  g1_ref[...] = jnp.concatenate(
      [dinv_ref[0:NN][:, None] * h, zt], axis=0)


def _tc_mid_body(sp_ref, h1_ref, b1_ref, w2_ref, b2_ref, wfc_ref, bfc_ref,
                 dinv_ref, p_ref, q_ref, us_ref, ut_ref):
  dinv = dinv_ref[0:NN][:, None]
  ssum = sp_ref[0, 0:NN, :] + sp_ref[1, 0:NN, :]
  z1 = jnp.maximum(
      dinv * ssum + dinv * dinv * h1_ref[0:NN, :] + b1_ref[...], 0.0)
  h2 = jnp.dot(z1, w2_ref[...], preferred_element_type=jnp.float32)
  u2 = dinv * dinv * h2 + b2_ref[...]
  wa = wfc_ref[0:HH, 0]
  wb = wfc_ref[HH:2 * HH, 0]
  p = dinv[:, 0] * jnp.dot(h2, wa, preferred_element_type=jnp.float32)
  q = dinv[:, 0] * jnp.dot(h2, wb, preferred_element_type=jnp.float32)
  ztail = jnp.zeros((NP - NN,), jnp.float32)
  p_ref[...] = jnp.concatenate([p, ztail])
  q_ref[...] = jnp.concatenate([q, ztail])
  us_ref[...] = jnp.concatenate(
      [jnp.dot(u2, wa, preferred_element_type=jnp.float32) + bfc_ref[...],
       ztail])
  ut_ref[...] = jnp.concatenate(
      [jnp.dot(u2, wb, preferred_element_type=jnp.float32), ztail])


_tc_h1 = pl.pallas_call(
    _tc_h1_body,
    out_shape=[jax.ShapeDtypeStruct((NP, HH), jnp.float32),
               jax.ShapeDtypeStruct((NP, HH), jnp.float32)],
)

_tc_mid = pl.pallas_call(
    _tc_mid_body,
    out_shape=[jax.ShapeDtypeStruct((NP,), jnp.float32),
               jax.ShapeDtypeStruct((NP,), jnp.float32),
               jax.ShapeDtypeStruct((NP,), jnp.float32),
               jax.ShapeDtypeStruct((NP,), jnp.float32)],
)


def kernel(x, edge_index, W1, b1, W2, b2, Wfc, bfc):
  src = edge_index[0]
  dst = edge_index[1]
  npad = EPAD - EE
  srcp = jnp.concatenate([src, jnp.zeros((npad,), jnp.int32)])
  dstp = jnp.concatenate([dst, jnp.full((npad,), NN, jnp.int32)])
  src3 = srcp.reshape(NWORK, NCH, CHUNK)
  dst3 = dstp.reshape(NWORK, NCH, CHUNK)

  zeros64 = jnp.zeros((NP, HH), jnp.float32)

  dinv_p = _sc_degree(dst3)
  h1p, g1p = _tc_h1(x, W1, dinv_p)
  s1_part = _sc_msgpass(g1p, src3, dst3, zeros64)
  p_p, q_p, us_p, ut_p = _tc_mid(s1_part, h1p, b1, W2, b2, Wfc, bfc, dinv_p)
  dec = _sc_scalar_decode(p_p, q_p, src3, dst3, dinv_p, us_p, ut_p)
  return dec.reshape(EPAD)[:EE].reshape(EE, 1)


# speedup vs baseline: 1.0121x; 1.0121x over previous
import functools

import jax
import jax.numpy as jnp
from jax import lax
from jax.experimental import pallas as pl
from jax.experimental.pallas import tpu as pltpu
from jax.experimental.pallas import tpu_sc as plsc

NN = 10000
EE = 320000
DD = 128
HH = 64
NC, NS, LL = 2, 16, 16
NWORK = NC * NS
CHUNK = 128
NCH = 80
NBUF = 2
NGRP = NCH // NBUF
EPAD = NWORK * NCH * CHUNK
ROWS_PER_TILE = 632
NP = NS * ROWS_PER_TILE
EW = EE // NWORK

_MESH = plsc.VectorSubcoreMesh(
    core_axis_name="c", subcore_axis_name="s", num_cores=NC, num_subcores=NS)


def _wid():
  return lax.axis_index("c") * NS + lax.axis_index("s")


@functools.partial(
    pl.kernel,
    out_type=jax.ShapeDtypeStruct((NP,), jnp.float32),
    mesh=_MESH,
    compiler_params=pltpu.CompilerParams(
        use_tc_tiling_on_sc=False, needs_layout_passes=False),
    scratch_types=[
        pltpu.VMEM((2 * NCH, CHUNK), jnp.int32),
        pltpu.VMEM((NP,), jnp.float32),
        pltpu.VMEM((640,), jnp.float32),
        pltpu.VMEM((640,), jnp.float32),
        pltpu.VMEM((640,), jnp.float32),
        pltpu.VMEM_SHARED((NS, NP), jnp.float32),
    ],
)
def _sc_degree(dst3, dinv_out, dst_l, cnt_l, dinv_l, accb, tbuf, stage):
  c = lax.axis_index("c")
  s = lax.axis_index("s")
  rbase = s * ROWS_PER_TILE
  rpt = pl.ds(rbase, ROWS_PER_TILE)
  pltpu.sync_copy(dst3.at[s], dst_l.at[pl.ds(0, NCH)])
  pltpu.sync_copy(dst3.at[s + NS], dst_l.at[pl.ds(NCH, NCH)])

  @pl.loop(0, NP // LL)
  def _(i):
    cnt_l[pl.ds(i * LL, LL)] = jnp.zeros((LL,), jnp.float32)

  ones = jnp.ones((LL,), jnp.float32)

  @pl.loop(0, 2 * NCH)
  def _(j):
    for k in range(CHUNK // LL):
      dv = dst_l[j, pl.ds(k * LL, LL)]
      plsc.addupdate_scatter(cnt_l, [dv], ones)

  pltpu.sync_copy(cnt_l, stage.at[s])
  plsc.subcore_barrier()

  @pl.loop(0, 640 // LL)
  def _(i):
    accb[pl.ds(i * LL, LL)] = jnp.zeros((LL,), jnp.float32)

  for t in range(NS):
    pltpu.sync_copy(stage.at[t, rpt], tbuf.at[pl.ds(0, ROWS_PER_TILE)])

    @pl.loop(0, 640 // LL)
    def _(i):
      sl = pl.ds(i * LL, LL)
      accb[sl] = accb[sl] + tbuf[sl]

  @pl.loop(0, 640 // LL)
  def _(i):
    sl = pl.ds(i * LL, LL)
    d = accb[sl] + 1.0
    bits = plsc.bitcast(d, jnp.int32)
    y = plsc.bitcast(0x5F3759DF - lax.shift_right_logical(bits, 1),
                     jnp.float32)
    y = y * (1.5 - 0.5 * d * y * y)
    y = y * (1.5 - 0.5 * d * y * y)
    y = y * (1.5 - 0.5 * d * y * y)
    dinv_l[sl] = y

  @pl.when(c == 0)
  def _():
    pltpu.sync_copy(dinv_l.at[pl.ds(0, ROWS_PER_TILE)], dinv_out.at[rpt])


@functools.partial(
    pl.kernel,
    out_type=jax.ShapeDtypeStruct((NC, NP, HH), jnp.float32),
    mesh=_MESH,
    compiler_params=pltpu.CompilerParams(
        use_tc_tiling_on_sc=False, needs_layout_passes=False),
    scratch_types=[
        pltpu.VMEM((NCH, CHUNK), jnp.int32),
        pltpu.VMEM((NCH, CHUNK), jnp.int32),
        pltpu.VMEM((CHUNK, HH), jnp.float32),
        pltpu.VMEM((CHUNK, HH), jnp.float32),
        pltpu.SemaphoreType.DMA,
        pltpu.SemaphoreType.DMA,
        pltpu.VMEM((640,), jnp.float32),
        pltpu.VMEM_SHARED((NP, HH), jnp.float32),
        pltpu.VMEM_SHARED((NP, HH), jnp.float32),
    ],
)
def _sc_msgpass(h1p, src3, dst3, zeros64, dinv_p, s_out,
                src_l, dst_l, rows0, rows1, sem0, sem1, dinv_l, g_sp, acc):
  c = lax.axis_index("c")
  s = lax.axis_index("s")
  w = _wid()
  rbase = s * ROWS_PER_TILE
  rpt = pl.ds(rbase, ROWS_PER_TILE)
  pltpu.sync_copy(src3.at[w], src_l)
  pltpu.sync_copy(dst3.at[w], dst_l)
  pltpu.sync_copy(dinv_p.at[rpt], dinv_l.at[pl.ds(0, ROWS_PER_TILE)])
  pltpu.sync_copy(zeros64.at[rpt], acc.at[rpt])

  for ch in range(5):
    n = 128 if ch < 4 else ROWS_PER_TILE - 512
    off = ch * 128
    pltpu.sync_copy(h1p.at[pl.ds(rbase + off, n)], rows0.at[pl.ds(0, n)])

    @pl.loop(0, n)
    def _(r):
      dvr = plsc.load_gather(dinv_l, [jnp.full((LL,), off, jnp.int32) + r])
      for kk in range(HH // LL):
        sl = pl.ds(kk * LL, LL)
        rows0[r, sl] = dvr * rows0[r, sl]

    pltpu.sync_copy(rows0.at[pl.ds(0, n)], g_sp.at[pl.ds(rbase + off, n)])

  plsc.subcore_barrier()

  pltpu.async_copy(g_sp.at[src_l.at[0]], rows0, sem0)

  @pl.loop(0, NCH - 1)
  def _(j):
    even = j % 2 == 0

    def do(cur, nxt, sem_cur, sem_nxt):
      pltpu.async_copy(g_sp.at[src_l.at[j + 1]], nxt, sem_nxt)
      pltpu.make_async_copy(g_sp.at[src_l.at[j]], cur, sem_cur).wait()
      pltpu.sync_copy(cur, acc.at[dst_l.at[j]], add=True)

    @pl.when(even)
    def _():
      do(rows0, rows1, sem0, sem1)

    @pl.when(jnp.logical_not(even))
    def _():
      do(rows1, rows0, sem1, sem0)

  last = NCH - 1
  pltpu.make_async_copy(g_sp.at[src_l.at[last]], rows1, sem1).wait()
  pltpu.sync_copy(rows1, acc.at[dst_l.at[last]], add=True)

  plsc.subcore_barrier()
  pltpu.sync_copy(acc.at[rpt], s_out.at[c, rpt])


@functools.partial(
    pl.kernel,
    out_type=jax.ShapeDtypeStruct((NWORK, NCH * CHUNK), jnp.float32),
    mesh=_MESH,
    compiler_params=pltpu.CompilerParams(
        use_tc_tiling_on_sc=False, needs_layout_passes=False),
    scratch_types=[
        pltpu.VMEM((2 * NCH, CHUNK), jnp.int32),
        pltpu.VMEM((2 * NCH, CHUNK), jnp.int32),
        pltpu.VMEM((NP,), jnp.float32),
        pltpu.VMEM((NP,), jnp.float32),
        pltpu.VMEM((NP,), jnp.float32),
        pltpu.VMEM((NP,), jnp.float32),
        pltpu.VMEM((640,), jnp.float32),
        pltpu.VMEM((640,), jnp.float32),
        pltpu.VMEM((640,), jnp.float32),
        pltpu.VMEM((640,), jnp.float32),
        pltpu.VMEM((640,), jnp.float32),
        pltpu.VMEM((640,), jnp.float32),
        pltpu.VMEM((NP,), jnp.float32),
        pltpu.VMEM((NP,), jnp.float32),
        pltpu.VMEM((NCH * CHUNK,), jnp.float32),
        pltpu.VMEM_SHARED((NS, NP), jnp.float32),
        pltpu.VMEM_SHARED((NP,), jnp.float32),
        pltpu.VMEM_SHARED((NP,), jnp.float32),
    ],
)
def _sc_scalar_decode(p_p, q_p, src3, dst3, dinv_p, us_p, ut_p,
                      dec_out,
                      src_l, dst_l, p_l, q_l, pacc, qacc,
                      dinv_l, us_l, ut_l, accb, tbuf, sv_l,
                      s_l, t_l, ob,
                      stage, s_sp, t_sp):
  c = lax.axis_index("c")
  s = lax.axis_index("s")
  rbase = s * ROWS_PER_TILE
  rpt = pl.ds(rbase, ROWS_PER_TILE)
  pltpu.sync_copy(p_p, p_l)
  pltpu.sync_copy(q_p, q_l)
  pltpu.sync_copy(src3.at[s], src_l.at[pl.ds(0, NCH)])
  pltpu.sync_copy(src3.at[s + NS], src_l.at[pl.ds(NCH, NCH)])
  pltpu.sync_copy(dst3.at[s], dst_l.at[pl.ds(0, NCH)])
  pltpu.sync_copy(dst3.at[s + NS], dst_l.at[pl.ds(NCH, NCH)])
  pltpu.sync_copy(dinv_p.at[rpt], dinv_l.at[pl.ds(0, ROWS_PER_TILE)])
  pltpu.sync_copy(us_p.at[rpt], us_l.at[pl.ds(0, ROWS_PER_TILE)])
  pltpu.sync_copy(ut_p.at[rpt], ut_l.at[pl.ds(0, ROWS_PER_TILE)])

  @pl.loop(0, NP // LL)
  def _(i):
    z = jnp.zeros((LL,), jnp.float32)
    pacc[pl.ds(i * LL, LL)] = z
    qacc[pl.ds(i * LL, LL)] = z

  @pl.loop(0, 2 * NCH)
  def _(j):
    for k in range(CHUNK // LL):
      sv = src_l[j, pl.ds(k * LL, LL)]
      dv = dst_l[j, pl.ds(k * LL, LL)]
      pv = plsc.load_gather(p_l, [sv])
      qv = plsc.load_gather(q_l, [sv])
      plsc.addupdate_scatter(pacc, [dv], pv)
      plsc.addupdate_scatter(qacc, [dv], qv)

  def reduce_into(ul, dst_sp):
    @pl.loop(0, 640 // LL)
    def _(i):
      accb[pl.ds(i * LL, LL)] = jnp.zeros((LL,), jnp.float32)

    for t in range(NS):
      pltpu.sync_copy(stage.at[t, rpt], tbuf.at[pl.ds(0, ROWS_PER_TILE)])

      @pl.loop(0, 640 // LL)
      def _(i):
        sl = pl.ds(i * LL, LL)
        accb[sl] = accb[sl] + tbuf[sl]

    @pl.loop(0, 640 // LL)
    def _(i):
      sl = pl.ds(i * LL, LL)
      sv_l[sl] = dinv_l[sl] * accb[sl] + ul[sl]

    pltpu.sync_copy(sv_l.at[pl.ds(0, ROWS_PER_TILE)], dst_sp.at[rpt])

  pltpu.sync_copy(pacc, stage.at[s])
  plsc.subcore_barrier()
  reduce_into(us_l, s_sp)
  plsc.subcore_barrier()
  pltpu.sync_copy(qacc, stage.at[s])
  plsc.subcore_barrier()
  reduce_into(ut_l, t_sp)
  plsc.subcore_barrier()

  pltpu.sync_copy(s_sp, s_l)
  pltpu.sync_copy(t_sp, t_l)
  w = _wid()
  wrow = c * NCH

  @pl.loop(0, NCH)
  def _(j):
    for k in range(CHUNK // LL):
      si = src_l[wrow + j, pl.ds(k * LL, LL)]
      di = dst_l[wrow + j, pl.ds(k * LL, LL)]
      sv = plsc.load_gather(s_l, [si])
      tv = plsc.load_gather(t_l, [di])
      y = sv + tv
      ob[pl.ds(j * CHUNK + k * LL, LL)] = 1.0 / (1.0 + jnp.exp(-y))

  pltpu.sync_copy(ob, dec_out.at[w])


def _tc_h1_body(x_ref, w1_ref, h1_ref):
  h = jnp.dot(x_ref[...], w1_ref[...], preferred_element_type=jnp.float32)
  h1_ref[...] = jnp.concatenate(
      [h, jnp.zeros((NP - NN, HH), jnp.float32)], axis=0)


def _tc_mid_body(sp_ref, h1_ref, b1_ref, w2_ref, b2_ref, wfc_ref, bfc_ref,
                 dinv_ref, p_ref, q_ref, us_ref, ut_ref):
  dinv = dinv_ref[0:NN][:, None]
  ssum = sp_ref[0, 0:NN, :] + sp_ref[1, 0:NN, :]
  z1 = jnp.maximum(
      dinv * ssum + dinv * dinv * h1_ref[0:NN, :] + b1_ref[...], 0.0)
  h2 = jnp.dot(z1, w2_ref[...], preferred_element_type=jnp.float32)
  u2 = dinv * dinv * h2 + b2_ref[...]
  wa = wfc_ref[0:HH, 0]
  wb = wfc_ref[HH:2 * HH, 0]
  p = dinv[:, 0] * jnp.dot(h2, wa, preferred_element_type=jnp.float32)
  q = dinv[:, 0] * jnp.dot(h2, wb, preferred_element_type=jnp.float32)
  ztail = jnp.zeros((NP - NN,), jnp.float32)
  p_ref[...] = jnp.concatenate([p, ztail])
  q_ref[...] = jnp.concatenate([q, ztail])
  us_ref[...] = jnp.concatenate(
      [jnp.dot(u2, wa, preferred_element_type=jnp.float32) + bfc_ref[...],
       ztail])
  ut_ref[...] = jnp.concatenate(
      [jnp.dot(u2, wb, preferred_element_type=jnp.float32), ztail])


_tc_h1 = pl.pallas_call(
    _tc_h1_body,
    out_shape=jax.ShapeDtypeStruct((NP, HH), jnp.float32),
)

_tc_mid = pl.pallas_call(
    _tc_mid_body,
    out_shape=[jax.ShapeDtypeStruct((NP,), jnp.float32),
               jax.ShapeDtypeStruct((NP,), jnp.float32),
               jax.ShapeDtypeStruct((NP,), jnp.float32),
               jax.ShapeDtypeStruct((NP,), jnp.float32)],
)


def kernel(x, edge_index, W1, b1, W2, b2, Wfc, bfc):
  src = edge_index[0]
  dst = edge_index[1]
  npad = EPAD - EE
  srcp = jnp.concatenate([src, jnp.zeros((npad,), jnp.int32)])
  dstp = jnp.concatenate([dst, jnp.full((npad,), NN, jnp.int32)])
  src3 = srcp.reshape(NWORK, NCH, CHUNK)
  dst3 = dstp.reshape(NWORK, NCH, CHUNK)

  zeros64 = jnp.zeros((NP, HH), jnp.float32)

  h1p = _tc_h1(x, W1)
  dinv_p = _sc_degree(dst3)
  s1_part = _sc_msgpass(h1p, src3, dst3, zeros64, dinv_p)
  p_p, q_p, us_p, ut_p = _tc_mid(s1_part, h1p, b1, W2, b2, Wfc, bfc, dinv_p)
  dec = _sc_scalar_decode(p_p, q_p, src3, dst3, dinv_p, us_p, ut_p)
  return dec.reshape(EPAD)[:EE].reshape(EE, 1)
